# baseline (speedup 1.0000x reference)
import functools

import jax
import jax.numpy as jnp
from jax import lax
from jax.experimental import pallas as pl
from jax.experimental.pallas import tpu as pltpu
from jax.experimental.pallas import tpu_sc as plsc

_L = 16


def _sc_topk_body(x_hbm, out_hbm, xloc, bits_ref, hist_ref, merged_ref,
                  mask_ref, hsh0, hsh1, hsh2, hsh3, *, topk, cols, bsz):
    sid = lax.axis_index("s")
    cpt = cols // _L
    nsl = cpt // _L
    base = sid * cpt
    hshs = (hsh0, hsh1, hsh2, hsh3)

    pltpu.sync_copy(x_hbm.at[:, pl.ds(base, cpt)], xloc)

    lanes = lax.iota(jnp.int32, _L)
    ones = jnp.ones((_L,), jnp.int32)
    zeros_i = jnp.zeros((_L,), jnp.int32)

    inv = jnp.float32(1.0 / bsz)
    for k in range(nsl):
        acc = jnp.zeros((_L,), jnp.float32)
        for r in range(bsz):
            acc = acc + jnp.abs(xloc[r, pl.ds(k * _L, _L)])
        bits_ref[pl.ds(k * _L, _L)] = lax.bitcast_convert_type(
            acc * inv, jnp.int32)

    prefix = jnp.int32(0)
    rank = jnp.int32(topk)
    for p, (sh, w) in enumerate(((23, 8), (15, 8), (7, 8), (0, 7))):
        dmask = jnp.int32((1 << w) - 1)
        hi = sh + w
        for j in range(_L):
            hist_ref[pl.ds(j * _L, _L)] = zeros_i
        for k in range(nsl):
            b = bits_ref[pl.ds(k * _L, _L)]
            active = (b >> hi) == (prefix >> hi)
            digit = (b >> sh) & dmask
            plsc.addupdate_scatter(hist_ref, [digit], ones, mask=active)
        pltpu.sync_copy(hist_ref, hshs[p].at[sid])
        plsc.subcore_barrier()
        pltpu.sync_copy(hshs[p], merged_ref)

        hsum = []
        for j in range(_L):
            h = jnp.zeros((_L,), jnp.int32)
            for t in range(_L):
                h = h + merged_ref[t, pl.ds(j * _L, _L)]
            hsum.append(h)
        tvec = jnp.zeros((_L,), jnp.int32)
        for j in range(_L):
            tvec = jnp.where(lanes == j, jnp.sum(hsum[j]), tvec)
        csuf = jnp.flip(jnp.cumsum(jnp.flip(tvec)))
        cond = csuf >= rank
        jstar = jnp.max(plsc.all_reduce_population_count(cond)) - 1
        rank_in = rank - jnp.sum(jnp.where(cond, 0, tvec))
        hsel = jnp.zeros((_L,), jnp.int32)
        for j in range(_L):
            hsel = jnp.where(jstar == j, hsum[j], hsel)
        lsuf = jnp.flip(jnp.cumsum(jnp.flip(hsel)))
        cond2 = lsuf >= rank_in
        lstar = jnp.max(plsc.all_reduce_population_count(cond2)) - 1
        rank = rank_in - jnp.sum(jnp.where(cond2, 0, hsel))
        prefix = prefix | ((jstar * _L + lstar) << sh)

    for k in range(nsl):
        keep = bits_ref[pl.ds(k * _L, _L)] >= prefix
        mask_ref[pl.ds(k * _L, _L)] = jnp.where(keep, 1.0, 0.0)
    pltpu.sync_copy(mask_ref, out_hbm.at[pl.ds(base, cpt)])


def _sc_topk_mask(x2, topk):
    bsz, cols = x2.shape
    mesh = plsc.VectorSubcoreMesh(
        core_axis_name="c", subcore_axis_name="s", num_cores=1)
    cpt = cols // _L
    f = pl.kernel(
        functools.partial(_sc_topk_body, topk=topk, cols=cols, bsz=bsz),
        out_type=jax.ShapeDtypeStruct((cols,), jnp.float32),
        mesh=mesh,
        compiler_params=pltpu.CompilerParams(needs_layout_passes=False),
        scratch_types=[
            pltpu.VMEM((bsz, cpt), jnp.float32),
            pltpu.VMEM((cpt,), jnp.int32),
            pltpu.VMEM((256,), jnp.int32),
            pltpu.VMEM((_L, 256), jnp.int32),
            pltpu.VMEM((cpt,), jnp.float32),
            pltpu.VMEM_SHARED((_L, 256), jnp.int32),
            pltpu.VMEM_SHARED((_L, 256), jnp.int32),
            pltpu.VMEM_SHARED((_L, 256), jnp.int32),
            pltpu.VMEM_SHARED((_L, 256), jnp.int32),
        ],
    )
    return f(x2)


def _tc_matmul_body(x_ref, m_ref, w_ref, b_ref, o_ref, xm_ref):
    i = pl.program_id(0)

    @pl.when(i == 0)
    def _():
        xm_ref[...] = x_ref[...] * m_ref[...]

    acc = jax.lax.dot_general(
        xm_ref[...], w_ref[...],
        (((1,), (1,)), ((), ())),
        preferred_element_type=jnp.float32,
    )
    o_ref[...] = acc + b_ref[...]


def kernel(x, weight, bias):
    bsz, seq, in_f = x.shape
    out_f = weight.shape[0]
    topk = int(in_f * 0.1)
    block_r = 1024

    x2 = x.reshape(bsz * seq, in_f)
    b2 = bias.reshape(1, out_f)

    mask = _sc_topk_mask(x2, topk).reshape(1, in_f)

    out = pl.pallas_call(
        _tc_matmul_body,
        grid=(pl.cdiv(out_f, block_r),),
        in_specs=[
            pl.BlockSpec((bsz * seq, in_f), lambda i: (0, 0)),
            pl.BlockSpec((1, in_f), lambda i: (0, 0)),
            pl.BlockSpec((block_r, in_f), lambda i: (i, 0)),
            pl.BlockSpec((1, block_r), lambda i: (0, i)),
        ],
        out_specs=pl.BlockSpec((bsz * seq, block_r), lambda i: (0, i)),
        out_shape=jax.ShapeDtypeStruct((bsz * seq, out_f), jnp.float32),
        scratch_shapes=[pltpu.VMEM((bsz * seq, in_f), jnp.float32)],
    )(x2, mask, weight, b2)
    return out.reshape(bsz, seq, out_f)


# speedup vs baseline: 1.0464x; 1.0464x over previous
import functools

import jax
import jax.numpy as jnp
from jax import lax
from jax.experimental import pallas as pl
from jax.experimental.pallas import tpu as pltpu
from jax.experimental.pallas import tpu_sc as plsc

_L = 16


def _sc_topk_body(x_hbm, out_hbm, xloc, bits_ref, hist_ref, merged_ref,
                  mask_ref, idx0_ref, idx1_ref, hsh0, hsh1, hsh2, hsh3,
                  *, topk, cols, bsz):
    sid = lax.axis_index("s")
    cpt = cols // _L
    nsl = cpt // _L
    base = sid * cpt
    hshs = (hsh0, hsh1, hsh2, hsh3)

    pltpu.sync_copy(x_hbm.at[:, pl.ds(base, cpt)], xloc)

    lanes = lax.iota(jnp.int32, _L)
    ones = jnp.ones((_L,), jnp.int32)
    zeros_i = jnp.zeros((_L,), jnp.int32)
    for k in range(8):
        idx0_ref[pl.ds(k * _L, _L)] = lanes + k * _L
        idx1_ref[pl.ds(k * _L, _L)] = lanes + k * _L + 128

    inv = jnp.float32(1.0 / bsz)
    for k in range(nsl):
        acc = jnp.zeros((_L,), jnp.float32)
        for r in range(bsz):
            acc = acc + jnp.abs(xloc[r, pl.ds(k * _L, _L)])
        bits_ref[pl.ds(k * _L, _L)] = lax.bitcast_convert_type(
            acc * inv, jnp.int32)

    @pl.when(sid == 0)
    def _zero_shared():
        for j in range(_L):
            hist_ref[pl.ds(j * _L, _L)] = zeros_i
        for h in hshs:
            pltpu.sync_copy(hist_ref, h)

    plsc.subcore_barrier()

    prefix = jnp.int32(0)
    rank = jnp.int32(topk)
    for p, (sh, w) in enumerate(((23, 8), (15, 8), (7, 8), (0, 7))):
        dmask = jnp.int32((1 << w) - 1)
        hi = sh + w
        for j in range(_L):
            hist_ref[pl.ds(j * _L, _L)] = zeros_i
        for k in range(nsl):
            b = bits_ref[pl.ds(k * _L, _L)]
            active = (b >> hi) == (prefix >> hi)
            digit = (b >> sh) & dmask
            plsc.addupdate_scatter(hist_ref, [digit], ones, mask=active)
        pltpu.sync_copy(hist_ref.at[pl.ds(0, 128)],
                        hshs[p].at[idx0_ref], add=True)
        pltpu.sync_copy(hist_ref.at[pl.ds(128, 128)],
                        hshs[p].at[idx1_ref], add=True)
        plsc.subcore_barrier()
        pltpu.sync_copy(hshs[p], merged_ref)

        hsum = [merged_ref[pl.ds(j * _L, _L)] for j in range(_L)]
        tvec = jnp.zeros((_L,), jnp.int32)
        for j in range(_L):
            tvec = jnp.where(lanes == j, jnp.sum(hsum[j]), tvec)
        csuf = jnp.flip(jnp.cumsum(jnp.flip(tvec)))
        cond = csuf >= rank
        jstar = jnp.max(plsc.all_reduce_population_count(cond)) - 1
        rank_in = rank - jnp.sum(jnp.where(cond, 0, tvec))
        hsel = jnp.zeros((_L,), jnp.int32)
        for j in range(_L):
            hsel = jnp.where(jstar == j, hsum[j], hsel)
        lsuf = jnp.flip(jnp.cumsum(jnp.flip(hsel)))
        cond2 = lsuf >= rank_in
        lstar = jnp.max(plsc.all_reduce_population_count(cond2)) - 1
        rank = rank_in - jnp.sum(jnp.where(cond2, 0, hsel))
        prefix = prefix | ((jstar * _L + lstar) << sh)

    for k in range(nsl):
        keep = bits_ref[pl.ds(k * _L, _L)] >= prefix
        mask_ref[pl.ds(k * _L, _L)] = jnp.where(keep, 1.0, 0.0)
    pltpu.sync_copy(mask_ref, out_hbm.at[pl.ds(base, cpt)])


def _sc_topk_mask(x2, topk):
    bsz, cols = x2.shape
    mesh = plsc.VectorSubcoreMesh(
        core_axis_name="c", subcore_axis_name="s", num_cores=1)
    cpt = cols // _L
    f = pl.kernel(
        functools.partial(_sc_topk_body, topk=topk, cols=cols, bsz=bsz),
        out_type=jax.ShapeDtypeStruct((cols,), jnp.float32),
        mesh=mesh,
        compiler_params=pltpu.CompilerParams(needs_layout_passes=False),
        scratch_types=[
            pltpu.VMEM((bsz, cpt), jnp.float32),
            pltpu.VMEM((cpt,), jnp.int32),
            pltpu.VMEM((256,), jnp.int32),
            pltpu.VMEM((256,), jnp.int32),
            pltpu.VMEM((cpt,), jnp.float32),
            pltpu.VMEM((128,), jnp.int32),
            pltpu.VMEM((128,), jnp.int32),
            pltpu.VMEM_SHARED((256,), jnp.int32),
            pltpu.VMEM_SHARED((256,), jnp.int32),
            pltpu.VMEM_SHARED((256,), jnp.int32),
            pltpu.VMEM_SHARED((256,), jnp.int32),
        ],
    )
    return f(x2)


def _tc_matmul_body(x_ref, m_ref, w_ref, b_ref, o_ref, xm_ref):
    i = pl.program_id(0)

    @pl.when(i == 0)
    def _():
        xm_ref[...] = x_ref[...] * m_ref[...]

    acc = jax.lax.dot_general(
        xm_ref[...], w_ref[...],
        (((1,), (1,)), ((), ())),
        preferred_element_type=jnp.float32,
    )
    o_ref[...] = acc + b_ref[...]


def kernel(x, weight, bias):
    bsz, seq, in_f = x.shape
    out_f = weight.shape[0]
    topk = int(in_f * 0.1)
    block_r = 1024

    x2 = x.reshape(bsz * seq, in_f)
    b2 = bias.reshape(1, out_f)

    mask = _sc_topk_mask(x2, topk).reshape(1, in_f)

    out = pl.pallas_call(
        _tc_matmul_body,
        grid=(pl.cdiv(out_f, block_r),),
        in_specs=[
            pl.BlockSpec((bsz * seq, in_f), lambda i: (0, 0)),
            pl.BlockSpec((1, in_f), lambda i: (0, 0)),
            pl.BlockSpec((block_r, in_f), lambda i: (i, 0)),
            pl.BlockSpec((1, block_r), lambda i: (0, i)),
        ],
        out_specs=pl.BlockSpec((bsz * seq, block_r), lambda i: (0, i)),
        out_shape=jax.ShapeDtypeStruct((bsz * seq, out_f), jnp.float32),
        scratch_shapes=[pltpu.VMEM((bsz * seq, in_f), jnp.float32)],
    )(x2, mask, weight, b2)
    return out.reshape(bsz, seq, out_f)
